# baseline (speedup 1.0000x reference)
import functools

import jax
import jax.numpy as jnp
from jax import lax
from jax.experimental import pallas as pl
from jax.experimental.pallas import tpu as pltpu
from jax.experimental.pallas import tpu_sc as plsc

_ALPHA = 0.05
_W = 1733
_U = 173


def _loss_kernel(x_ref, t_ref, out_ref, m_run, se_run, sx_run):
    i = pl.program_id(0)
    nb = pl.num_programs(0)
    V = nb * _W
    neg_inf = jnp.full(t_ref.shape, -jnp.inf, jnp.float32)
    zero = jnp.zeros(t_ref.shape, jnp.float32)

    @pl.when(i == 0)
    def _init():
        m_run[...] = neg_inf
        se_run[...] = zero
        sx_run[...] = zero


    def p1(j, carry):
        m0, m1, s0, s1 = carry
        for u in range(_U):
            c = x_ref[j * _U + u]
            if u % 2 == 0:
                m0 = jnp.maximum(m0, c)
                s0 = s0 + c
            else:
                m1 = jnp.maximum(m1, c)
                s1 = s1 + c
        return m0, m1, s0, s1

    m0, m1, s0, s1 = jax.lax.fori_loop(
        0, _W // _U, p1, (neg_inf, neg_inf, zero, zero))
    for k in range((_W // _U) * _U, _W):
        c = x_ref[k]
        m0 = jnp.maximum(m0, c)
        s0 = s0 + c
    bm = jnp.maximum(m0, m1)
    m_old = m_run[...]
    m_new = jnp.maximum(m_old, bm)

    def p2(j, carry):
        e0, e1 = carry
        for u in range(_U):
            c = x_ref[j * _U + u]
            e = jnp.exp(c - m_new)
            if u % 2 == 0:
                e0 = e0 + e
            else:
                e1 = e1 + e
        return e0, e1

    e0, e1 = jax.lax.fori_loop(0, _W // _U, p2, (zero, zero))
    for k in range((_W // _U) * _U, _W):
        e0 = e0 + jnp.exp(x_ref[k] - m_new)

    m_run[...] = m_new
    se_run[...] = se_run[...] * jnp.exp(m_old - m_new) + (e0 + e1)
    sx_run[...] = sx_run[...] + (s0 + s1)

    @pl.when(i == nb - 1)
    def _fin():
        raw = t_ref[...]
        ts = jnp.concatenate(
            [raw[:, 1:], jnp.zeros((raw.shape[0], 1), jnp.int32)], axis=1)
        lse = m_run[...] + jnp.log(se_run[...])
        base = (_ALPHA / V) * sx_run[...] - lse
        mask = ts != 0
        out_ref[0, 0] = jnp.sum(jnp.where(mask, base, 0.0))
        out_ref[0, 1] = jnp.sum(jnp.where(mask, 1.0, 0.0))


def _sc_gather_call(x1, t_flat, n_pos):
    info = plsc.get_sparse_core_info()
    nw = info.num_cores * info.num_subcores
    per_w = n_pos // nw
    mesh = plsc.VectorSubcoreMesh(core_axis_name="c", subcore_axis_name="s")

    @functools.partial(
        pl.kernel,
        mesh=mesh,
        out_type=jax.ShapeDtypeStruct((nw, per_w), jnp.float32),
        scratch_types=[
            pltpu.VMEM((per_w,), jnp.int32),
            pltpu.VMEM((per_w,), jnp.int32),
            pltpu.VMEM((per_w,), jnp.float32),
            pltpu.VMEM((per_w,), jnp.float32),
            pltpu.SemaphoreType.DMA,
        ],
    )
    def sc_gather(x_hbm, t_hbm, out_hbm, tv, eidx, vals, ov, sem):
        wid = lax.axis_index("s") * info.num_cores + lax.axis_index("c")
        base = wid * per_w
        pltpu.sync_copy(t_hbm.at[pl.ds(base, per_w)], tv)
        for c2 in range(per_w // 16):
            rr = lax.iota(jnp.int32, 16) + (c2 * 16)
            tvc = tv[pl.ds(c2 * 16, 16)]
            eidx[pl.ds(c2 * 16, 16)] = tvc * 1024 + rr + base
        pltpu.async_copy(x_hbm.at[eidx], vals, sem).wait()
        for c2 in range(per_w // 16):
            tvc = tv[pl.ds(c2 * 16, 16)]
            g16 = vals[pl.ds(c2 * 16, 16)]
            ov[pl.ds(c2 * 16, 16)] = jnp.where(tvc != 0, g16, 0.0)
        pltpu.sync_copy(ov, out_hbm.at[wid])

    return sc_gather(x1, t_flat)


def kernel(model_out, tgt):
    S, B, V = model_out.shape
    xt = jnp.transpose(model_out, (2, 1, 0))
    tgt = tgt.astype(jnp.int32)
    t_shift = jnp.roll(tgt, -1, axis=1).at[:, -1].set(0)
    x1 = jnp.reshape(xt, (V * B * S,))
    sc_out = _sc_gather_call(x1, t_shift.reshape(-1), B * S)
    nb = V // _W
    tc_out = pl.pallas_call(
        _loss_kernel,
        grid=(nb,),
        in_specs=[
            pl.BlockSpec((_W, B, S), lambda i: (i, 0, 0)),
            pl.BlockSpec((B, S), lambda i: (0, 0)),
        ],
        out_specs=pl.BlockSpec(memory_space=pltpu.SMEM),
        out_shape=jax.ShapeDtypeStruct((1, 2), jnp.float32),
        scratch_shapes=[pltpu.VMEM((B, S), jnp.float32) for _ in range(3)],
        compiler_params=pltpu.CompilerParams(dimension_semantics=("arbitrary",)),
    )(xt, tgt)
    g_sum = jnp.sum(sc_out)
    return -(tc_out[0, 0] + (1.0 - _ALPHA) * g_sum) / tc_out[0, 1]


# speedup vs baseline: 1.2928x; 1.2928x over previous
import jax
import jax.numpy as jnp
from jax.experimental import pallas as pl
from jax.experimental.pallas import tpu as pltpu

_ALPHA = 0.05
_W = 1733
_U = 173


def _loss_kernel(x_ref, t_ref, out_ref, ts_ref, m_run, se_run, sx_run, gx_run):
    i = pl.program_id(0)
    nb = pl.num_programs(0)
    V = nb * _W
    neg_inf = jnp.full(t_ref.shape, -jnp.inf, jnp.float32)
    zero = jnp.zeros(t_ref.shape, jnp.float32)

    @pl.when(i == 0)
    def _init():
        raw = t_ref[...]
        ts_ref[...] = jnp.concatenate(
            [raw[:, 1:], jnp.zeros((raw.shape[0], 1), jnp.int32)], axis=1)
        m_run[...] = neg_inf
        se_run[...] = zero
        sx_run[...] = zero
        gx_run[...] = zero

    t = ts_ref[...]
    t_rel = t - i * _W

    def p1(j, carry):
        m0, m1, s0, s1, g0, g1 = carry
        tg = t_rel - j * _U
        for u in range(_U):
            c = x_ref[j * _U + u]
            sel = jnp.where(tg == u, c, 0.0)
            if u % 2 == 0:
                m0 = jnp.maximum(m0, c)
                s0 = s0 + c
                g0 = g0 + sel
            else:
                m1 = jnp.maximum(m1, c)
                s1 = s1 + c
                g1 = g1 + sel
        return m0, m1, s0, s1, g0, g1

    m0, m1, s0, s1, g0, g1 = jax.lax.fori_loop(
        0, _W // _U, p1, (neg_inf, neg_inf, zero, zero, zero, zero))
    for k in range((_W // _U) * _U, _W):
        c = x_ref[k]
        m0 = jnp.maximum(m0, c)
        s0 = s0 + c
        g0 = g0 + jnp.where(t_rel == k, c, 0.0)
    bm = jnp.maximum(m0, m1)
    m_old = m_run[...]
    m_new = jnp.maximum(m_old, bm)

    def p2(j, carry):
        e0, e1 = carry
        for u in range(_U):
            c = x_ref[j * _U + u]
            e = jnp.exp(c - m_new)
            if u % 2 == 0:
                e0 = e0 + e
            else:
                e1 = e1 + e
        return e0, e1

    e0, e1 = jax.lax.fori_loop(0, _W // _U, p2, (zero, zero))
    for k in range((_W // _U) * _U, _W):
        e0 = e0 + jnp.exp(x_ref[k] - m_new)

    m_run[...] = m_new
    se_run[...] = se_run[...] * jnp.exp(m_old - m_new) + (e0 + e1)
    sx_run[...] = sx_run[...] + (s0 + s1)
    gx_run[...] = gx_run[...] + (g0 + g1)

    @pl.when(i == nb - 1)
    def _fin():
        lse = m_run[...] + jnp.log(se_run[...])
        base = (_ALPHA / V) * sx_run[...] - lse + (1.0 - _ALPHA) * gx_run[...]
        mask = t != 0
        contrib = jnp.sum(jnp.where(mask, base, 0.0))
        cnt = jnp.sum(jnp.where(mask, 1.0, 0.0))
        out_ref[0, 0] = -contrib / cnt


def kernel(model_out, tgt):
    S, B, V = model_out.shape
    xt = jnp.transpose(model_out, (2, 1, 0))
    tgt = tgt.astype(jnp.int32)
    nb = V // _W
    out = pl.pallas_call(
        _loss_kernel,
        grid=(nb,),
        in_specs=[
            pl.BlockSpec((_W, B, S), lambda i: (i, 0, 0)),
            pl.BlockSpec((B, S), lambda i: (0, 0)),
        ],
        out_specs=pl.BlockSpec(memory_space=pltpu.SMEM),
        out_shape=jax.ShapeDtypeStruct((1, 1), jnp.float32),
        scratch_shapes=[pltpu.VMEM((B, S), jnp.int32)]
        + [pltpu.VMEM((B, S), jnp.float32) for _ in range(4)],
        compiler_params=pltpu.CompilerParams(dimension_semantics=("arbitrary",)),
    )(xt, tgt)
    return out[0, 0]
